# baseline (speedup 1.0000x reference)
import functools

import jax
import jax.numpy as jnp
from jax.experimental import pallas as pl
from jax.experimental.pallas import tpu as pltpu

_PLACEHOLDER_ID = 12345


def _masked_copy_body(tok_ref, emb_ref, ph_ref, out_ref):
    bb, S, D = out_ref.shape
    tok3 = jax.lax.broadcast_in_dim(tok_ref[...], (bb, S, D), (0, 1))
    mask = tok3 == _PLACEHOLDER_ID
    out_ref[...] = jnp.where(mask, ph_ref[...][None, None, :], emb_ref[...])


def kernel(tokenized_text, embedded_text, placeholder_emb):
    B, S, D = embedded_text.shape
    tok = tokenized_text.astype(jnp.int32)
    bb = 32
    grid = (B // bb,)
    out = pl.pallas_call(
        _masked_copy_body,
        grid=grid,
        in_specs=[
            pl.BlockSpec((bb, S), lambda i: (i, 0)),
            pl.BlockSpec((bb, S, D), lambda i: (i, 0, 0)),
            pl.BlockSpec((D,), lambda i: (0,)),
        ],
        out_specs=pl.BlockSpec((bb, S, D), lambda i: (i, 0, 0)),
        out_shape=jax.ShapeDtypeStruct((B, S, D), jnp.float32),
        compiler_params=pltpu.CompilerParams(
            dimension_semantics=("arbitrary",),
        ),
    )(tok, embedded_text, placeholder_emb)
    return out


# speedup vs baseline: 1.3603x; 1.3603x over previous
import functools

import jax
import jax.numpy as jnp
from jax import lax
from jax.experimental import pallas as pl
from jax.experimental.pallas import tpu as pltpu
from jax.experimental.pallas import tpu_sc as plsc

_PLACEHOLDER_ID = 12345
_NW = 32
_L = 16


def kernel(tokenized_text, embedded_text, placeholder_emb):
    B, S, D = embedded_text.shape
    N = B * S
    per_w = N // _NW
    ngroups = per_w // _L
    tok = tokenized_text.reshape(N).astype(jnp.int32)
    mesh = plsc.VectorSubcoreMesh(core_axis_name="c", subcore_axis_name="s")

    @functools.partial(
        pl.kernel,
        out_type=(),
        mesh=mesh,
        compiler_params=pltpu.CompilerParams(needs_layout_passes=False),
        scratch_types=[
            pltpu.VMEM((per_w,), jnp.int32),
            pltpu.VMEM((D,), jnp.float32),
        ],
    )
    def run(tok_hbm, ph_hbm, buf, tok_v, ph_v):
        wid = lax.axis_index("s") * 2 + lax.axis_index("c")
        base = wid * per_w
        pltpu.sync_copy(tok_hbm.at[pl.ds(base, per_w)], tok_v)
        pltpu.sync_copy(ph_hbm, ph_v)
        lanes = lax.iota(jnp.int32, _L)

        def group(g, _):
            tokv = tok_v[pl.ds(g * _L, _L)]
            m0 = tokv == _PLACEHOLDER_ID
            cnt = plsc.all_reduce_population_count(m0)
            cnt_s = lax.reduce_max(cnt, (0,))

            @pl.when(cnt_s > 0)
            def _scatter():
                for l in range(_L):
                    t_l = lax.reduce_max(
                        jnp.where(lanes == l, tokv, jnp.int32(0)), (0,)
                    )

                    @pl.when(t_l == _PLACEHOLDER_ID)
                    def _one():
                        row = base + g * _L + l
                        b = row // S
                        s = row - b * S
                        pltpu.sync_copy(ph_v, buf.at[b, s])

            return _

        lax.fori_loop(0, ngroups, group, None)

    buf = jax.new_ref(embedded_text)
    run(tok, placeholder_emb, buf)
    return buf[...]
